# baseline (speedup 1.0000x reference)
import functools

import jax
import jax.numpy as jnp
from jax import lax
from jax.experimental import pallas as pl
from jax.experimental.pallas import tpu as pltpu
from jax.experimental.pallas import tpu_sc as plsc

N_NODES = 10000
N_EDGES = 320000
D_IN = 128
H2 = 256
H = 128

NC = 2
NS = 16
NW = NC * NS
EPW = N_EDGES // NW
K = 64
NB = 158
EPAD = NW * NB * K
NPAD = 10240
RPS = NPAD // NS
CPAD = NPAD
CPS = CPAD // NS

_HIGH = jax.lax.Precision.HIGHEST


def _seg_sum_sc(feats, ei4, z2d, z1d, ones, with_counts):
    D = feats.shape[1]
    mesh = plsc.VectorSubcoreMesh(core_axis_name="c", subcore_axis_name="s")
    out_type = [jax.ShapeDtypeStruct((NC, NPAD, D), jnp.float32)]
    scratch = [
        pltpu.VMEM_SHARED((NPAD, D), jnp.float32),
        pltpu.VMEM((8, 2, K), jnp.int32),
        pltpu.VMEM((4, K, D), jnp.float32),
        pltpu.SemaphoreType.DMA((8,)),
        pltpu.SemaphoreType.DMA((4,)),
        pltpu.SemaphoreType.DMA((4,)),
    ]
    if with_counts:
        out_type.append(jax.ShapeDtypeStruct((NC, CPAD), jnp.float32))
        scratch += [
            pltpu.VMEM_SHARED((CPAD,), jnp.float32),
            pltpu.VMEM((K,), jnp.float32),
        ]

    def body(x_hbm, ei_hbm, z2_hbm, z1_hbm, ones_hbm, *rest):
        if with_counts:
            (agg_hbm, cnt_hbm, acc_sp, ibuf, rows_v, isems, gsems, ssems,
             cnt_sp, ones_v) = rest
        else:
            agg_hbm, acc_sp, ibuf, rows_v, isems, gsems, ssems = rest
        rows = tuple(rows_v.at[i] for i in range(4))
        c = lax.axis_index("c")
        s_ = lax.axis_index("s")
        w = s_ * NC + c
        pltpu.sync_copy(z2_hbm, acc_sp.at[pl.ds(s_ * RPS, RPS)])
        if with_counts:
            pltpu.sync_copy(z1_hbm, cnt_sp.at[pl.ds(s_ * CPS, CPS)])
            pltpu.sync_copy(ones_hbm, ones_v)
        plsc.subcore_barrier()

        def fetch(jb, islot):
            pltpu.async_copy(ei_hbm.at[w, jb], ibuf.at[islot],
                             isems.at[islot])

        def gather(jb, islot, b):
            pltpu.make_async_copy(ei_hbm.at[w, jb], ibuf.at[islot],
                                  isems.at[islot]).wait()
            pltpu.async_copy(x_hbm.at[ibuf.at[islot, 0]], rows[b],
                             gsems.at[b])

        def drain_scatter(b):
            pltpu.make_async_copy(rows[b], acc_sp.at[ibuf.at[0, 1]],
                                  ssems.at[b]).wait()

        def slot(j, b, i, do_drain, do_fetch, do_gather):
            pltpu.make_async_copy(x_hbm.at[ibuf.at[i, 0]], rows[b],
                                  gsems.at[b]).wait()
            pltpu.async_copy(rows[b], acc_sp.at[ibuf.at[i, 1]],
                             ssems.at[b], add=True)
            if with_counts:
                pltpu.sync_copy(ones_v, cnt_sp.at[ibuf.at[i, 1]], add=True)
            if do_drain:
                drain_scatter((b + 2) % 4)
            if do_fetch:
                fetch(j + 5, (i + 5) % 8)
            if do_gather:
                gather(j + 2, (i + 2) % 8, (b + 2) % 4)

        for islot in range(5):
            fetch(islot, islot)
        gather(0, 0, 0)
        gather(1, 1, 1)
        for t in range(8):
            slot(t, t % 4, t % 8, t >= 2, True, True)

        @pl.loop(8, 152, step=8)
        def _(j):
            for r in range(8):
                slot(j + r, r % 4, r, True, True, True)

        for t in range(152, NB):
            slot(t, t % 4, t % 8, True, t + 5 < NB, t + 2 < NB)
        drain_scatter((NB - 2) % 4)
        drain_scatter((NB - 1) % 4)

        plsc.subcore_barrier()
        pltpu.sync_copy(acc_sp.at[pl.ds(s_ * RPS, RPS)],
                        agg_hbm.at[c, pl.ds(s_ * RPS, RPS)])
        if with_counts:
            pltpu.sync_copy(cnt_sp.at[pl.ds(s_ * CPS, CPS)],
                            cnt_hbm.at[c, pl.ds(s_ * CPS, CPS)])

    return pl.kernel(body, out_type=tuple(out_type), mesh=mesh,
                     scratch_types=scratch)(feats, ei4, z2d, z1d, ones)


R = 2000
GRID = N_NODES // R


def _tc1_body(a_ref, cnt_ref, x_ref, w1l_ref, b1_ref, w1r_ref, w2l_ref,
              w2r_ref, b2_ref, g_ref, r_ref):
    a = a_ref[0] + a_ref[1]
    cnt = cnt_ref[:, 0:1] + cnt_ref[:, 1:2]
    inv = 1.0 / jnp.maximum(cnt, 1.0)
    mean = a * inv
    t = (jnp.dot(mean, w1l_ref[...], precision=_HIGH,
                 preferred_element_type=jnp.float32)
         + jnp.dot(x_ref[...], w1r_ref[...], precision=_HIGH,
                   preferred_element_type=jnp.float32)
         + b1_ref[...])
    h = jnp.maximum(t, 0.0)
    g_ref[...] = jnp.dot(h, w2l_ref[...], precision=_HIGH,
                         preferred_element_type=jnp.float32)
    r_ref[...] = jnp.dot(h, w2r_ref[...], precision=_HIGH,
                         preferred_element_type=jnp.float32) + b2_ref[...]


def _tc1(agg1, cnt2, x, w1l_t, b1, w1r_t, w2l_t, w2r_t, b2):
    return pl.pallas_call(
        _tc1_body,
        grid=(GRID,),
        in_specs=[
            pl.BlockSpec((NC, R, D_IN), lambda i: (0, i, 0)),
            pl.BlockSpec((R, NC), lambda i: (i, 0)),
            pl.BlockSpec((R, D_IN), lambda i: (i, 0)),
            pl.BlockSpec((D_IN, H2), lambda i: (0, 0)),
            pl.BlockSpec((1, H2), lambda i: (0, 0)),
            pl.BlockSpec((D_IN, H2), lambda i: (0, 0)),
            pl.BlockSpec((H2, H), lambda i: (0, 0)),
            pl.BlockSpec((H2, H), lambda i: (0, 0)),
            pl.BlockSpec((1, H), lambda i: (0, 0)),
        ],
        out_specs=[
            pl.BlockSpec((R, H), lambda i: (i, 0)),
            pl.BlockSpec((R, H), lambda i: (i, 0)),
        ],
        out_shape=[
            jax.ShapeDtypeStruct((N_NODES, H), jnp.float32),
            jax.ShapeDtypeStruct((N_NODES, H), jnp.float32),
        ],
    )(agg1, cnt2, x, w1l_t, b1, w1r_t, w2l_t, w2r_t, b2)


def _tc2_body(a_ref, cnt_ref, r_ref, o_ref):
    a = a_ref[0] + a_ref[1]
    cnt = cnt_ref[:, 0:1] + cnt_ref[:, 1:2]
    inv = 1.0 / jnp.maximum(cnt, 1.0)
    t = a * inv + r_ref[...]
    m = jnp.max(t, axis=1, keepdims=True)
    e = jnp.exp(t - m)
    lse = jnp.log(jnp.sum(e, axis=1, keepdims=True))
    o_ref[...] = t - m - lse


def _tc2(agg2, cnt2, r):
    return pl.pallas_call(
        _tc2_body,
        grid=(GRID,),
        in_specs=[
            pl.BlockSpec((NC, R, H), lambda i: (0, i, 0)),
            pl.BlockSpec((R, NC), lambda i: (i, 0)),
            pl.BlockSpec((R, H), lambda i: (i, 0)),
        ],
        out_specs=pl.BlockSpec((R, H), lambda i: (i, 0)),
        out_shape=jax.ShapeDtypeStruct((N_NODES, H), jnp.float32),
    )(agg2, cnt2, r)


def kernel(x, edge_index, W1_l, b1_l, W1_r, W2_l, b2_l, W2_r):
    npad = EPAD - N_EDGES
    src = jnp.concatenate([edge_index[0].astype(jnp.int32),
                           jnp.zeros((npad,), jnp.int32)])
    dst = jnp.concatenate([edge_index[1].astype(jnp.int32),
                           jnp.full((npad,), N_NODES, jnp.int32)])
    ei4 = jnp.stack([src.reshape(NW, NB, K), dst.reshape(NW, NB, K)], axis=2)
    z2d = jnp.zeros((RPS, D_IN), jnp.float32)
    z1d = jnp.zeros((CPS,), jnp.float32)
    ones = jnp.ones((K,), jnp.float32)

    agg1, cnt = _seg_sum_sc(x, ei4, z2d, z1d, ones, with_counts=True)
    cnt2 = cnt[:, :N_NODES].T

    g, r = _tc1(agg1, cnt2, x,
                W1_l.T, b1_l.reshape(1, H2), W1_r.T,
                W2_l.T, W2_r.T, b2_l.reshape(1, H))

    (agg2,) = _seg_sum_sc(g, ei4, z2d, z1d, ones, with_counts=False)
    return _tc2(agg2, cnt2, r)


# speedup vs baseline: 1.4009x; 1.4009x over previous
import functools

import jax
import jax.numpy as jnp
from jax import lax
from jax.experimental import pallas as pl
from jax.experimental.pallas import tpu as pltpu
from jax.experimental.pallas import tpu_sc as plsc

N_NODES = 10000
N_EDGES = 320000
D_IN = 128
H2 = 256
H = 128

NC = 2
NS = 16
NW = NC * NS
EPW = N_EDGES // NW
K = 125
NB = EPW // K
NPAD = 10240
RPS = NPAD // NS
CPAD = NPAD
CPS = CPAD // NS

_HIGH = jax.lax.Precision.HIGHEST


def _seg_sum_sc(feats, src3, dst3, z2d, z1d, ones, with_counts):
    D = feats.shape[1]
    mesh = plsc.VectorSubcoreMesh(core_axis_name="c", subcore_axis_name="s")
    out_type = [jax.ShapeDtypeStruct((NC, NPAD, D), jnp.float32)]
    scratch = [
        pltpu.VMEM_SHARED((NPAD, D), jnp.float32),
        pltpu.VMEM((2, NB, K), jnp.int32),
        pltpu.VMEM((K, D), jnp.float32),
    ]
    if with_counts:
        out_type.append(jax.ShapeDtypeStruct((NC, CPAD), jnp.float32))
        scratch += [
            pltpu.VMEM_SHARED((CPAD,), jnp.float32),
            pltpu.VMEM((K,), jnp.float32),
        ]

    def body(x_hbm, src_hbm, dst_hbm, z2_hbm, z1_hbm, ones_hbm, *rest):
        if with_counts:
            (agg_hbm, cnt_hbm, acc_sp, idx_v, rows_v, cnt_sp, ones_v) = rest
        else:
            agg_hbm, acc_sp, idx_v, rows_v = rest
        c = lax.axis_index("c")
        s_ = lax.axis_index("s")
        w = s_ * NC + c
        pltpu.sync_copy(z2_hbm, acc_sp.at[pl.ds(s_ * RPS, RPS)])
        pltpu.sync_copy(src_hbm.at[w], idx_v.at[0])
        pltpu.sync_copy(dst_hbm.at[w], idx_v.at[1])
        if with_counts:
            pltpu.sync_copy(z1_hbm, cnt_sp.at[pl.ds(s_ * CPS, CPS)])
            pltpu.sync_copy(ones_hbm, ones_v)
        plsc.subcore_barrier()

        @pl.loop(0, NB)
        def _(j):
            pltpu.sync_copy(x_hbm.at[idx_v.at[0, j]], rows_v)
            pltpu.sync_copy(rows_v, acc_sp.at[idx_v.at[1, j]], add=True)
            if with_counts:
                pltpu.sync_copy(ones_v, cnt_sp.at[idx_v.at[1, j]], add=True)

        plsc.subcore_barrier()
        pltpu.sync_copy(acc_sp.at[pl.ds(s_ * RPS, RPS)],
                        agg_hbm.at[c, pl.ds(s_ * RPS, RPS)])
        if with_counts:
            pltpu.sync_copy(cnt_sp.at[pl.ds(s_ * CPS, CPS)],
                            cnt_hbm.at[c, pl.ds(s_ * CPS, CPS)])

    return pl.kernel(body, out_type=tuple(out_type), mesh=mesh,
                     scratch_types=scratch)(feats, src3, dst3, z2d, z1d,
                                             ones)


R = 2000
GRID = N_NODES // R


def _tc1_body(a_ref, cnt_ref, x_ref, w1l_ref, b1_ref, w1r_ref, w2l_ref,
              w2r_ref, b2_ref, g_ref, r_ref):
    a = a_ref[0] + a_ref[1]
    cnt = cnt_ref[:, 0:1] + cnt_ref[:, 1:2]
    inv = 1.0 / jnp.maximum(cnt, 1.0)
    mean = a * inv
    t = (jnp.dot(mean, w1l_ref[...], precision=_HIGH,
                 preferred_element_type=jnp.float32)
         + jnp.dot(x_ref[...], w1r_ref[...], precision=_HIGH,
                   preferred_element_type=jnp.float32)
         + b1_ref[...])
    h = jnp.maximum(t, 0.0)
    g_ref[...] = jnp.dot(h, w2l_ref[...], precision=_HIGH,
                         preferred_element_type=jnp.float32)
    r_ref[...] = jnp.dot(h, w2r_ref[...], precision=_HIGH,
                         preferred_element_type=jnp.float32) + b2_ref[...]


def _tc1(agg1, cnt2, x, w1l_t, b1, w1r_t, w2l_t, w2r_t, b2):
    return pl.pallas_call(
        _tc1_body,
        grid=(GRID,),
        in_specs=[
            pl.BlockSpec((NC, R, D_IN), lambda i: (0, i, 0)),
            pl.BlockSpec((R, NC), lambda i: (i, 0)),
            pl.BlockSpec((R, D_IN), lambda i: (i, 0)),
            pl.BlockSpec((D_IN, H2), lambda i: (0, 0)),
            pl.BlockSpec((1, H2), lambda i: (0, 0)),
            pl.BlockSpec((D_IN, H2), lambda i: (0, 0)),
            pl.BlockSpec((H2, H), lambda i: (0, 0)),
            pl.BlockSpec((H2, H), lambda i: (0, 0)),
            pl.BlockSpec((1, H), lambda i: (0, 0)),
        ],
        out_specs=[
            pl.BlockSpec((R, H), lambda i: (i, 0)),
            pl.BlockSpec((R, H), lambda i: (i, 0)),
        ],
        out_shape=[
            jax.ShapeDtypeStruct((N_NODES, H), jnp.float32),
            jax.ShapeDtypeStruct((N_NODES, H), jnp.float32),
        ],
    )(agg1, cnt2, x, w1l_t, b1, w1r_t, w2l_t, w2r_t, b2)


def _tc2_body(a_ref, cnt_ref, r_ref, o_ref):
    a = a_ref[0] + a_ref[1]
    cnt = cnt_ref[:, 0:1] + cnt_ref[:, 1:2]
    inv = 1.0 / jnp.maximum(cnt, 1.0)
    t = a * inv + r_ref[...]
    m = jnp.max(t, axis=1, keepdims=True)
    e = jnp.exp(t - m)
    lse = jnp.log(jnp.sum(e, axis=1, keepdims=True))
    o_ref[...] = t - m - lse


def _tc2(agg2, cnt2, r):
    return pl.pallas_call(
        _tc2_body,
        grid=(GRID,),
        in_specs=[
            pl.BlockSpec((NC, R, H), lambda i: (0, i, 0)),
            pl.BlockSpec((R, NC), lambda i: (i, 0)),
            pl.BlockSpec((R, H), lambda i: (i, 0)),
        ],
        out_specs=pl.BlockSpec((R, H), lambda i: (i, 0)),
        out_shape=jax.ShapeDtypeStruct((N_NODES, H), jnp.float32),
    )(agg2, cnt2, r)


def kernel(x, edge_index, W1_l, b1_l, W1_r, W2_l, b2_l, W2_r):
    src3 = edge_index[0].astype(jnp.int32).reshape(NW, NB, K)
    dst3 = edge_index[1].astype(jnp.int32).reshape(NW, NB, K)
    z2d = jnp.zeros((RPS, D_IN), jnp.float32)
    z1d = jnp.zeros((CPS,), jnp.float32)
    ones = jnp.ones((K,), jnp.float32)

    agg1, cnt = _seg_sum_sc(x, src3, dst3, z2d, z1d, ones,
                            with_counts=True)
    cnt2 = cnt[:, :N_NODES].T

    g, r = _tc1(agg1, cnt2, x,
                W1_l.T, b1_l.reshape(1, H2), W1_r.T,
                W2_l.T, W2_r.T, b2_l.reshape(1, H))

    (agg2,) = _seg_sum_sc(g, src3, dst3, z2d, z1d, ones,
                          with_counts=False)
    return _tc2(agg2, cnt2, r)


# speedup vs baseline: 1.9915x; 1.4216x over previous
import functools

import jax
import jax.numpy as jnp
from jax import lax
from jax.experimental import pallas as pl
from jax.experimental.pallas import tpu as pltpu
from jax.experimental.pallas import tpu_sc as plsc

N_NODES = 10000
N_EDGES = 320000
D_IN = 128
H2 = 256
H = 128

NC = 2
NS = 16
NW = NC * NS
EPW = N_EDGES // NW
K = 125
NB = EPW // K
NPAD = 10240
RPS = NPAD // NS
CPAD = NPAD
CPS = CPAD // NS

_HIGH = jax.lax.Precision.HIGHEST


def _seg_sum_sc(feats, src3, dst3, z2d, z1d, ones, with_counts):
    D = feats.shape[1]
    mesh = plsc.VectorSubcoreMesh(core_axis_name="c", subcore_axis_name="s")
    out_type = [jax.ShapeDtypeStruct((NC, NPAD, D), jnp.float32)]
    scratch = [
        pltpu.VMEM_SHARED((NPAD, D), jnp.float32),
        pltpu.VMEM((NB, K), jnp.int32),
        pltpu.VMEM((8, K), jnp.int32),
        pltpu.VMEM((2, K, D), jnp.float32),
        pltpu.SemaphoreType.DMA((8,)),
        pltpu.SemaphoreType.DMA((2,)),
    ]
    if with_counts:
        out_type.append(jax.ShapeDtypeStruct((NC, CPAD), jnp.float32))
        scratch += [
            pltpu.VMEM_SHARED((CPAD,), jnp.float32),
            pltpu.VMEM((K,), jnp.float32),
        ]

    def body(x_hbm, src_hbm, dst_hbm, z2_hbm, z1_hbm, ones_hbm, *rest):
        if with_counts:
            (agg_hbm, cnt_hbm, acc_sp, src_v, dring, rows_v, isems, gsems,
             cnt_sp, ones_v) = rest
        else:
            agg_hbm, acc_sp, src_v, dring, rows_v, isems, gsems = rest
        rows = (rows_v.at[0], rows_v.at[1])
        c = lax.axis_index("c")
        s_ = lax.axis_index("s")
        w = s_ * NC + c
        pltpu.sync_copy(z2_hbm, acc_sp.at[pl.ds(s_ * RPS, RPS)])
        pltpu.sync_copy(src_hbm.at[w], src_v)
        if with_counts:
            pltpu.sync_copy(z1_hbm, cnt_sp.at[pl.ds(s_ * CPS, CPS)])
            pltpu.sync_copy(ones_hbm, ones_v)
        plsc.subcore_barrier()

        def fetch_dst(jb, i):
            pltpu.async_copy(dst_hbm.at[w, jb], dring.at[i], isems.at[i])

        def gather(jb, b):
            pltpu.async_copy(x_hbm.at[src_v.at[jb]], rows[b], gsems.at[b])

        def slot(j, b, i, do_fetch, do_gather):
            pltpu.make_async_copy(x_hbm.at[src_v.at[j]], rows[b],
                                  gsems.at[b]).wait()
            pltpu.make_async_copy(dst_hbm.at[w, j], dring.at[i],
                                  isems.at[i]).wait()
            pltpu.sync_copy(rows[b], acc_sp.at[dring.at[i]], add=True)
            if with_counts:
                pltpu.sync_copy(ones_v, cnt_sp.at[dring.at[i]], add=True)
            if do_fetch:
                fetch_dst(j + 8, i)
            if do_gather:
                gather(j + 2, b)

        for i in range(8):
            fetch_dst(i, i)
        gather(0, 0)
        gather(1, 1)
        for t in range(8):
            slot(t, t % 2, t % 8, True, True)

        @pl.loop(8, 72, step=8)
        def _(j):
            for r in range(8):
                slot(j + r, r % 2, r, True, True)

        for t in range(72, NB):
            slot(t, t % 2, t % 8, False, t + 2 < NB)

        plsc.subcore_barrier()
        pltpu.sync_copy(acc_sp.at[pl.ds(s_ * RPS, RPS)],
                        agg_hbm.at[c, pl.ds(s_ * RPS, RPS)])
        if with_counts:
            pltpu.sync_copy(cnt_sp.at[pl.ds(s_ * CPS, CPS)],
                            cnt_hbm.at[c, pl.ds(s_ * CPS, CPS)])

    return pl.kernel(body, out_type=tuple(out_type), mesh=mesh,
                     scratch_types=scratch)(feats, src3, dst3, z2d, z1d,
                                             ones)


R = 2000
GRID = N_NODES // R


def _tc1_body(a_ref, cnt_ref, x_ref, w1l_ref, b1_ref, w1r_ref, w2l_ref,
              w2r_ref, b2_ref, g_ref, r_ref):
    a = a_ref[0] + a_ref[1]
    cnt = cnt_ref[:, 0:1] + cnt_ref[:, 1:2]
    inv = 1.0 / jnp.maximum(cnt, 1.0)
    mean = a * inv
    t = (jnp.dot(mean, w1l_ref[...], precision=_HIGH,
                 preferred_element_type=jnp.float32)
         + jnp.dot(x_ref[...], w1r_ref[...], precision=_HIGH,
                   preferred_element_type=jnp.float32)
         + b1_ref[...])
    h = jnp.maximum(t, 0.0)
    g_ref[...] = jnp.dot(h, w2l_ref[...], precision=_HIGH,
                         preferred_element_type=jnp.float32)
    r_ref[...] = jnp.dot(h, w2r_ref[...], precision=_HIGH,
                         preferred_element_type=jnp.float32) + b2_ref[...]


def _tc1(agg1, cnt2, x, w1l_t, b1, w1r_t, w2l_t, w2r_t, b2):
    return pl.pallas_call(
        _tc1_body,
        grid=(GRID,),
        in_specs=[
            pl.BlockSpec((NC, R, D_IN), lambda i: (0, i, 0)),
            pl.BlockSpec((R, NC), lambda i: (i, 0)),
            pl.BlockSpec((R, D_IN), lambda i: (i, 0)),
            pl.BlockSpec((D_IN, H2), lambda i: (0, 0)),
            pl.BlockSpec((1, H2), lambda i: (0, 0)),
            pl.BlockSpec((D_IN, H2), lambda i: (0, 0)),
            pl.BlockSpec((H2, H), lambda i: (0, 0)),
            pl.BlockSpec((H2, H), lambda i: (0, 0)),
            pl.BlockSpec((1, H), lambda i: (0, 0)),
        ],
        out_specs=[
            pl.BlockSpec((R, H), lambda i: (i, 0)),
            pl.BlockSpec((R, H), lambda i: (i, 0)),
        ],
        out_shape=[
            jax.ShapeDtypeStruct((N_NODES, H), jnp.float32),
            jax.ShapeDtypeStruct((N_NODES, H), jnp.float32),
        ],
    )(agg1, cnt2, x, w1l_t, b1, w1r_t, w2l_t, w2r_t, b2)


def _tc2_body(a_ref, cnt_ref, r_ref, o_ref):
    a = a_ref[0] + a_ref[1]
    cnt = cnt_ref[:, 0:1] + cnt_ref[:, 1:2]
    inv = 1.0 / jnp.maximum(cnt, 1.0)
    t = a * inv + r_ref[...]
    m = jnp.max(t, axis=1, keepdims=True)
    e = jnp.exp(t - m)
    lse = jnp.log(jnp.sum(e, axis=1, keepdims=True))
    o_ref[...] = t - m - lse


def _tc2(agg2, cnt2, r):
    return pl.pallas_call(
        _tc2_body,
        grid=(GRID,),
        in_specs=[
            pl.BlockSpec((NC, R, H), lambda i: (0, i, 0)),
            pl.BlockSpec((R, NC), lambda i: (i, 0)),
            pl.BlockSpec((R, H), lambda i: (i, 0)),
        ],
        out_specs=pl.BlockSpec((R, H), lambda i: (i, 0)),
        out_shape=jax.ShapeDtypeStruct((N_NODES, H), jnp.float32),
    )(agg2, cnt2, r)


def kernel(x, edge_index, W1_l, b1_l, W1_r, W2_l, b2_l, W2_r):
    src3 = edge_index[0].astype(jnp.int32).reshape(NW, NB, K)
    dst3 = edge_index[1].astype(jnp.int32).reshape(NW, NB, K)
    z2d = jnp.zeros((RPS, D_IN), jnp.float32)
    z1d = jnp.zeros((CPS,), jnp.float32)
    ones = jnp.ones((K,), jnp.float32)

    agg1, cnt = _seg_sum_sc(x, src3, dst3, z2d, z1d, ones,
                            with_counts=True)
    cnt2 = cnt[:, :N_NODES].T

    g, r = _tc1(agg1, cnt2, x,
                W1_l.T, b1_l.reshape(1, H2), W1_r.T,
                W2_l.T, W2_r.T, b2_l.reshape(1, H))

    (agg2,) = _seg_sum_sc(g, src3, dst3, z2d, z1d, ones,
                          with_counts=False)
    return _tc2(agg2, cnt2, r)


# speedup vs baseline: 2.0763x; 1.0426x over previous
import functools

import jax
import jax.numpy as jnp
from jax import lax
from jax.experimental import pallas as pl
from jax.experimental.pallas import tpu as pltpu
from jax.experimental.pallas import tpu_sc as plsc

N_NODES = 10000
N_EDGES = 320000
D_IN = 128
H2 = 256
H = 128

NC = 2
NS = 16
NW = NC * NS
EPW = N_EDGES // NW
K = 80
NB = EPW // K
NPAD = 10112
RPS = NPAD // NS
CPAD = 10240
CPS = CPAD // NS

_HIGH = jax.lax.Precision.HIGHEST


def _seg_sum_sc(feats, src3, dst3, z2d, z1d, ones, with_counts):
    D = feats.shape[1]
    mesh = plsc.VectorSubcoreMesh(core_axis_name="c", subcore_axis_name="s")
    out_type = [jax.ShapeDtypeStruct((NC, NPAD, D), jnp.float32)]
    scratch = [
        pltpu.VMEM_SHARED((NPAD, D), jnp.float32),
        pltpu.VMEM((128, K), jnp.int32),
        pltpu.VMEM((3, 4, K), jnp.int32),
        pltpu.VMEM((3, K, D), jnp.float32),
        pltpu.SemaphoreType.DMA((3,)),
        pltpu.SemaphoreType.DMA((3,)),
        pltpu.SemaphoreType.DMA((3,)),
    ]
    if with_counts:
        out_type.append(jax.ShapeDtypeStruct((NC, CPAD), jnp.float32))
        scratch += [
            pltpu.VMEM_SHARED((CPAD,), jnp.float32),
            pltpu.VMEM((K,), jnp.float32),
        ]

    def body(x_hbm, src_hbm, dst_hbm, z2_hbm, z1_hbm, ones_hbm, *rest):
        if with_counts:
            (agg_hbm, cnt_hbm, acc_sp, src_v, dring, rows_v, isems, gsems,
             ssems, cnt_sp, ones_v) = rest
        else:
            agg_hbm, acc_sp, src_v, dring, rows_v, isems, gsems, ssems = rest
        rows = tuple(rows_v.at[i] for i in range(3))
        c = lax.axis_index("c")
        s_ = lax.axis_index("s")
        w = s_ * NC + c
        pltpu.sync_copy(z2_hbm, acc_sp.at[pl.ds(s_ * RPS, RPS)])
        pltpu.sync_copy(src_hbm.at[w], src_v.at[pl.ds(0, NB)])
        if with_counts:
            pltpu.sync_copy(z1_hbm, cnt_sp.at[pl.ds(s_ * CPS, CPS)])
            pltpu.sync_copy(ones_hbm, ones_v)
        plsc.subcore_barrier()

        def fetch_set(jb, pos):
            pltpu.async_copy(dst_hbm.at[w, pl.ds(jb, 4)], dring.at[pos],
                             isems.at[pos])

        def gather(jb, b):
            pltpu.async_copy(x_hbm.at[src_v.at[jb]], rows[b], gsems.at[b])

        def drain_scatter(b):
            pltpu.make_async_copy(rows[b], acc_sp.at[dring.at[0, 0]],
                                  ssems.at[b]).wait()

        def slot(j, b, pos, q, first, do_fetch, do_gather):
            pltpu.make_async_copy(x_hbm.at[src_v.at[j]], rows[b],
                                  gsems.at[b]).wait()
            if q == 0:
                pltpu.make_async_copy(dst_hbm.at[w, pl.ds(j, 4)],
                                      dring.at[pos], isems.at[pos]).wait()
            if not first:
                drain_scatter((b + 2) % 3)
            pltpu.async_copy(rows[b], acc_sp.at[dring.at[pos, q]],
                             ssems.at[b], add=True)
            if with_counts:
                pltpu.sync_copy(ones_v, cnt_sp.at[dring.at[pos, q]],
                                add=True)
            if do_fetch:
                fetch_set(j + 8, (pos + 2) % 3)
            if do_gather:
                gather(j + 2, (b + 2) % 3)

        for pos in range(3):
            fetch_set(4 * pos, pos)
        gather(0, 0)
        gather(1, 1)
        for t in range(12):
            slot(t, t % 3, (t // 4) % 3, t % 4, t == 0, t % 4 == 0, True)

        @pl.loop(12, 120, step=12)
        def _(j):
            for r in range(12):
                slot(j + r, r % 3, (r // 4) % 3, r % 4, False,
                     r % 4 == 0, True)

        for t in range(120, NB):
            slot(t, t % 3, (t // 4) % 3, t % 4, False, False, t + 2 < NB)
        drain_scatter((NB - 1) % 3)

        plsc.subcore_barrier()
        pltpu.sync_copy(acc_sp.at[pl.ds(s_ * RPS, RPS)],
                        agg_hbm.at[c, pl.ds(s_ * RPS, RPS)])
        if with_counts:
            pltpu.sync_copy(cnt_sp.at[pl.ds(s_ * CPS, CPS)],
                            cnt_hbm.at[c, pl.ds(s_ * CPS, CPS)])

    return pl.kernel(body, out_type=tuple(out_type), mesh=mesh,
                     scratch_types=scratch)(feats, src3, dst3, z2d, z1d,
                                             ones)


R = 2000
GRID = N_NODES // R


def _tc1_body(a_ref, cnt_ref, x_ref, w1l_ref, b1_ref, w1r_ref, w2l_ref,
              w2r_ref, b2_ref, g_ref, r_ref):
    a = a_ref[0] + a_ref[1]
    cnt = cnt_ref[:, 0:1] + cnt_ref[:, 1:2]
    inv = 1.0 / jnp.maximum(cnt, 1.0)
    mean = a * inv
    t = (jnp.dot(mean, w1l_ref[...], precision=_HIGH,
                 preferred_element_type=jnp.float32)
         + jnp.dot(x_ref[...], w1r_ref[...], precision=_HIGH,
                   preferred_element_type=jnp.float32)
         + b1_ref[...])
    h = jnp.maximum(t, 0.0)
    g_ref[...] = jnp.dot(h, w2l_ref[...], precision=_HIGH,
                         preferred_element_type=jnp.float32)
    r_ref[...] = jnp.dot(h, w2r_ref[...], precision=_HIGH,
                         preferred_element_type=jnp.float32) + b2_ref[...]


def _tc1(agg1, cnt2, x, w1l_t, b1, w1r_t, w2l_t, w2r_t, b2):
    return pl.pallas_call(
        _tc1_body,
        grid=(GRID,),
        in_specs=[
            pl.BlockSpec((NC, R, D_IN), lambda i: (0, i, 0)),
            pl.BlockSpec((R, NC), lambda i: (i, 0)),
            pl.BlockSpec((R, D_IN), lambda i: (i, 0)),
            pl.BlockSpec((D_IN, H2), lambda i: (0, 0)),
            pl.BlockSpec((1, H2), lambda i: (0, 0)),
            pl.BlockSpec((D_IN, H2), lambda i: (0, 0)),
            pl.BlockSpec((H2, H), lambda i: (0, 0)),
            pl.BlockSpec((H2, H), lambda i: (0, 0)),
            pl.BlockSpec((1, H), lambda i: (0, 0)),
        ],
        out_specs=[
            pl.BlockSpec((R, H), lambda i: (i, 0)),
            pl.BlockSpec((R, H), lambda i: (i, 0)),
        ],
        out_shape=[
            jax.ShapeDtypeStruct((N_NODES, H), jnp.float32),
            jax.ShapeDtypeStruct((N_NODES, H), jnp.float32),
        ],
    )(agg1, cnt2, x, w1l_t, b1, w1r_t, w2l_t, w2r_t, b2)


def _tc2_body(a_ref, cnt_ref, r_ref, o_ref):
    a = a_ref[0] + a_ref[1]
    cnt = cnt_ref[:, 0:1] + cnt_ref[:, 1:2]
    inv = 1.0 / jnp.maximum(cnt, 1.0)
    t = a * inv + r_ref[...]
    m = jnp.max(t, axis=1, keepdims=True)
    e = jnp.exp(t - m)
    lse = jnp.log(jnp.sum(e, axis=1, keepdims=True))
    o_ref[...] = t - m - lse


def _tc2(agg2, cnt2, r):
    return pl.pallas_call(
        _tc2_body,
        grid=(GRID,),
        in_specs=[
            pl.BlockSpec((NC, R, H), lambda i: (0, i, 0)),
            pl.BlockSpec((R, NC), lambda i: (i, 0)),
            pl.BlockSpec((R, H), lambda i: (i, 0)),
        ],
        out_specs=pl.BlockSpec((R, H), lambda i: (i, 0)),
        out_shape=jax.ShapeDtypeStruct((N_NODES, H), jnp.float32),
    )(agg2, cnt2, r)


def kernel(x, edge_index, W1_l, b1_l, W1_r, W2_l, b2_l, W2_r):
    src3 = edge_index[0].astype(jnp.int32).reshape(NW, NB, K)
    dst3 = edge_index[1].astype(jnp.int32).reshape(NW, NB, K)
    dst3 = jnp.pad(dst3, ((0, 0), (0, 128 - NB), (0, 0)))
    z2d = jnp.zeros((RPS, D_IN), jnp.float32)
    z1d = jnp.zeros((CPS,), jnp.float32)
    ones = jnp.ones((K,), jnp.float32)

    agg1, cnt = _seg_sum_sc(x, src3, dst3, z2d, z1d, ones,
                            with_counts=True)
    cnt2 = cnt[:, :N_NODES].T

    g, r = _tc1(agg1, cnt2, x,
                W1_l.T, b1_l.reshape(1, H2), W1_r.T,
                W2_l.T, W2_r.T, b2_l.reshape(1, H))

    (agg2,) = _seg_sum_sc(g, src3, dst3, z2d, z1d, ones,
                          with_counts=False)
    return _tc2(agg2, cnt2, r)
